# Initial kernel scaffold; baseline (speedup 1.0000x reference)
import jax
import jax.numpy as jnp
from jax.experimental import pallas as pl


def kernel(x, edge, batch, W, b, gamma, beta):
    raise NotImplementedError("write your pallas kernel here")


# speedup vs baseline: 3.0984x; 3.0984x over previous
import functools

import jax
import jax.numpy as jnp
from jax.experimental import pallas as pl
from jax.experimental.pallas import tpu as pltpu

_SZ_C = 2
_SZ_L = 3


def _ln_kernel(h_ref, gamma_ref, beta_ref, o_ref):
    h = h_ref[...]
    mu = jnp.mean(h, axis=-1, keepdims=True)
    var = jnp.mean((h - mu) ** 2, axis=-1, keepdims=True)
    o_ref[...] = (h - mu) * jax.lax.rsqrt(var + 1e-6) * gamma_ref[...] + beta_ref[...]


def _layernorm(h, gamma, beta):
    C, N, D = h.shape
    BR = 400
    grid = (C, N // BR)
    return pl.pallas_call(
        _ln_kernel,
        grid=grid,
        in_specs=[
            pl.BlockSpec((1, BR, D), lambda c, i: (c, i, 0)),
            pl.BlockSpec((1, D), lambda c, i: (0, 0)),
            pl.BlockSpec((1, D), lambda c, i: (0, 0)),
        ],
        out_specs=pl.BlockSpec((1, BR, D), lambda c, i: (c, i, 0)),
        out_shape=jax.ShapeDtypeStruct((C, N, D), h.dtype),
    )(h, gamma[None], beta[None])


def kernel(x, edge, batch, W, b, gamma, beta):
    n = x.shape[0]
    src, dst = edge[0], edge[1]
    deg = jnp.ones((n,), jnp.float32).at[dst].add(1.0)
    dis = jax.lax.rsqrt(deg)

    def agg(hs):
        m = jnp.zeros_like(hs).at[dst].add(hs[src])
        return m + hs

    chans = []
    for c in range(_SZ_C):
        h = x
        for l in range(_SZ_L):
            hs = dis[:, None] * (h @ W[c, l])
            g = dis[:, None] * agg(hs) + b[c, l]
            h = jax.nn.relu(g) + g
        chans.append(h)
    channal = jnp.stack(chans, axis=0)
    ln = _layernorm(channal, gamma, beta)
    batchs = jnp.ones((_SZ_C, batch.shape[0]), dtype=x.dtype) * batch.astype(x.dtype)
    return (ln, batchs)


# speedup vs baseline: 17.8423x; 5.7586x over previous
import functools

import jax
import jax.numpy as jnp
from jax import lax
from jax.experimental import pallas as pl
from jax.experimental.pallas import tpu as pltpu
from jax.experimental.pallas import tpu_sc as plsc

N = 10000
NPAD = 10112
D = 128
E = 320000
EB = 128
NBLK1 = 80
NBLK2 = 160
G = 16
EPAD = 32 * NBLK1 * EB
ACCN = 10240
_SZ_C = 2
_SZ_L = 3


def _sc_mesh():
    return plsc.VectorSubcoreMesh(core_axis_name="c", subcore_axis_name="s")


def _make_deg_kernel():
    @functools.partial(
        pl.kernel,
        out_type=jax.ShapeDtypeStruct((2, ACCN), jnp.float32),
        mesh=_sc_mesh(),
        scratch_types=[
            pltpu.VMEM((NBLK1, EB), jnp.int32),
            pltpu.VMEM((EB,), jnp.float32),
            pltpu.VMEM((640,), jnp.float32),
            pltpu.VMEM_SHARED((ACCN,), jnp.float32),
        ],
    )
    def deg_kernel(dst_hbm, degp_hbm, dst_v, ones_v, z_v, acc):
        c = lax.axis_index("c")
        s = lax.axis_index("s")
        w = c * 16 + s
        for j in range(EB // 16):
            ones_v[pl.ds(j * 16, 16)] = jnp.ones((16,), jnp.float32)
        for j in range(640 // 16):
            z_v[pl.ds(j * 16, 16)] = jnp.zeros((16,), jnp.float32)
        pltpu.sync_copy(z_v, acc.at[pl.ds(s * 640, 640)])
        pltpu.sync_copy(dst_hbm.at[w], dst_v)
        plsc.subcore_barrier()

        def body(i, carry):
            pltpu.sync_copy(ones_v, acc.at[dst_v.at[i]], add=True)
            return carry

        lax.fori_loop(0, NBLK1, body, 0)
        plsc.subcore_barrier()
        pltpu.sync_copy(acc.at[pl.ds(s * 640, 640)],
                        degp_hbm.at[c, pl.ds(s * 640, 640)])

    return deg_kernel


def _make_agg_kernel(two_ch):
    nblk = NBLK2 if two_ch else NBLK1
    chunk = nblk * EB

    @functools.partial(
        pl.kernel,
        out_type=jax.ShapeDtypeStruct((2, NPAD, D), jnp.float32),
        mesh=_sc_mesh(),
        scratch_types=[
            pltpu.VMEM((G * EB,), jnp.int32),
            pltpu.VMEM((G, EB), jnp.int32),
            pltpu.VMEM((EB, D), jnp.float32),
            pltpu.VMEM_SHARED((NPAD, D), jnp.float32),
        ],
    )
    def agg_kernel(tbl_hbm, src_hbm, dst_hbm, out_hbm, src_g, dst_g, rows_v, acc):
        c = lax.axis_index("c")
        s = lax.axis_index("s")
        w = c * 16 + s

        def zbody(i, carry):
            for j in range(D // 16):
                rows_v[i, pl.ds(j * 16, 16)] = jnp.zeros((16,), jnp.float32)
            return carry

        lax.fori_loop(0, EB, zbody, 0)
        base = s * 632
        for k in range(4):
            pltpu.sync_copy(rows_v, acc.at[pl.ds(base + k * EB, EB)])
        pltpu.sync_copy(rows_v.at[pl.ds(0, 120)], acc.at[pl.ds(base + 4 * EB, 120)])
        plsc.subcore_barrier()

        def group(gi, carry):
            if two_ch:
                pltpu.sync_copy(src_hbm.at[c, pl.ds(s * chunk + gi * (G * EB), G * EB)], src_g)
                pltpu.sync_copy(dst_hbm.at[s, pl.ds(gi * G, G)], dst_g)
            else:
                pltpu.sync_copy(src_hbm.at[pl.ds(w * chunk + gi * (G * EB), G * EB)], src_g)
                pltpu.sync_copy(dst_hbm.at[w, pl.ds(gi * G, G)], dst_g)

            def blk(j, carry2):
                pltpu.sync_copy(tbl_hbm.at[src_g.at[pl.ds(j * EB, EB)]], rows_v)
                pltpu.sync_copy(rows_v, acc.at[dst_g.at[j]], add=True)
                return carry2

            lax.fori_loop(0, G, blk, 0)
            return carry

        lax.fori_loop(0, nblk // G, group, 0)
        plsc.subcore_barrier()
        dbase = s * 632
        pltpu.sync_copy(acc.at[pl.ds(dbase, 632)],
                        out_hbm.at[c, pl.ds(dbase, 632)])

    return agg_kernel


_deg_call = _make_deg_kernel()
_agg1_call = _make_agg_kernel(False)
_agg2_call = _make_agg_kernel(True)


_BRD = 2528
_BRL = 2528
_BRF = 2000


def _dis_body(degp_ref, x_ref, dis_ref, xs_ref):
    deg = degp_ref[:, 0] + degp_ref[:, 1] + 1.0
    dis = lax.rsqrt(deg)[:, None]
    dis_ref[...] = dis
    xs_ref[...] = dis * x_ref[...]


def _dis_call(degp, xp):
    grid = (NPAD // _BRD,)
    return pl.pallas_call(
        _dis_body,
        grid=grid,
        in_specs=[
            pl.BlockSpec((_BRD, 2), lambda i: (i, 0)),
            pl.BlockSpec((_BRD, D), lambda i: (i, 0)),
        ],
        out_specs=[
            pl.BlockSpec((_BRD, 1), lambda i: (i, 0)),
            pl.BlockSpec((_BRD, D), lambda i: (i, 0)),
        ],
        out_shape=[
            jax.ShapeDtypeStruct((NPAD, 1), jnp.float32),
            jax.ShapeDtypeStruct((NPAD, D), jnp.float32),
        ],
    )(degp, xp)


def _l0_body(y_ref, xs_ref, dis_ref, W_ref, b_ref, o_ref):
    dis = dis_ref[...]
    u = dis * (y_ref[0] + y_ref[1] + xs_ref[...])
    for c in range(_SZ_C):
        g = jnp.dot(u, W_ref[c], preferred_element_type=jnp.float32) + b_ref[c]
        h = jnp.maximum(g, 0.0) + g
        o_ref[c] = dis * h


def _l0_call(y, xs, dis, Wl, bl):
    grid = (NPAD // _BRL,)
    return pl.pallas_call(
        _l0_body,
        grid=grid,
        in_specs=[
            pl.BlockSpec((2, _BRL, D), lambda i: (0, i, 0)),
            pl.BlockSpec((_BRL, D), lambda i: (i, 0)),
            pl.BlockSpec((_BRL, 1), lambda i: (i, 0)),
            pl.BlockSpec((2, D, D), lambda i: (0, 0, 0)),
            pl.BlockSpec((2, 1, D), lambda i: (0, 0, 0)),
        ],
        out_specs=pl.BlockSpec((2, _BRL, D), lambda i: (0, i, 0)),
        out_shape=jax.ShapeDtypeStruct((2, NPAD, D), jnp.float32),
    )(y, xs, dis, Wl, bl)


def _mid_body(y_ref, hs_ref, dis_ref, W_ref, b_ref, o_ref):
    dis = dis_ref[...]
    for c in range(_SZ_C):
        v = dis * (y_ref[c] + hs_ref[c])
        g = jnp.dot(v, W_ref[c], preferred_element_type=jnp.float32) + b_ref[c]
        h = jnp.maximum(g, 0.0) + g
        o_ref[c] = dis * h


def _mid_call(y, hs, dis, Wl, bl):
    grid = (NPAD // _BRL,)
    return pl.pallas_call(
        _mid_body,
        grid=grid,
        in_specs=[
            pl.BlockSpec((2, _BRL, D), lambda i: (0, i, 0)),
            pl.BlockSpec((2, _BRL, D), lambda i: (0, i, 0)),
            pl.BlockSpec((_BRL, 1), lambda i: (i, 0)),
            pl.BlockSpec((2, D, D), lambda i: (0, 0, 0)),
            pl.BlockSpec((2, 1, D), lambda i: (0, 0, 0)),
        ],
        out_specs=pl.BlockSpec((2, _BRL, D), lambda i: (0, i, 0)),
        out_shape=jax.ShapeDtypeStruct((2, NPAD, D), jnp.float32),
    )(y, hs, dis, Wl, bl)


def _fin_body(y_ref, hs_ref, dis_ref, W_ref, b_ref, gamma_ref, beta_ref, o_ref):
    dis = dis_ref[...]
    for c in range(_SZ_C):
        v = dis * (y_ref[c] + hs_ref[c])
        g = jnp.dot(v, W_ref[c], preferred_element_type=jnp.float32) + b_ref[c]
        h = jnp.maximum(g, 0.0) + g
        mu = jnp.mean(h, axis=-1, keepdims=True)
        var = jnp.mean((h - mu) ** 2, axis=-1, keepdims=True)
        o_ref[c] = (h - mu) * lax.rsqrt(var + 1e-6) * gamma_ref[...] + beta_ref[...]


def _fin_call(y, hs, dis, Wl, bl, gamma, beta):
    grid = (N // _BRF,)
    return pl.pallas_call(
        _fin_body,
        grid=grid,
        in_specs=[
            pl.BlockSpec((2, _BRF, D), lambda i: (0, i, 0)),
            pl.BlockSpec((2, _BRF, D), lambda i: (0, i, 0)),
            pl.BlockSpec((_BRF, 1), lambda i: (i, 0)),
            pl.BlockSpec((2, D, D), lambda i: (0, 0, 0)),
            pl.BlockSpec((2, 1, D), lambda i: (0, 0, 0)),
            pl.BlockSpec((1, D), lambda i: (0, 0)),
            pl.BlockSpec((1, D), lambda i: (0, 0)),
        ],
        out_specs=pl.BlockSpec((2, _BRF, D), lambda i: (0, i, 0)),
        out_shape=jax.ShapeDtypeStruct((2, N, D), jnp.float32),
    )(y, hs, dis, Wl, bl, gamma, beta)


def kernel(x, edge, batch, W, b, gamma, beta):
    src, dst = edge[0], edge[1]
    npad_e = EPAD - E
    ar = jnp.arange(npad_e, dtype=jnp.int32)
    pad_src = (ar * 997) % N
    pad_dst = N + (ar % (NPAD - N))
    srcA = jnp.concatenate([src, pad_src])
    dstA = jnp.concatenate([dst, pad_dst])
    dst32 = dstA.reshape(32, NBLK1, EB)
    dst16 = dstA.reshape(16, NBLK2, EB)
    srcBC = jnp.stack([srcA, srcA + NPAD])
    xp = jnp.pad(x, ((0, NPAD - N), (0, 0)))

    degp = _deg_call(dst32)
    dis, xs = _dis_call(degp.T[:NPAD], xp)

    yA = _agg1_call(xs, srcA, dst32)
    HS1 = _l0_call(yA, xs, dis, W[:, 0], b[:, 0][:, None, :])
    y1 = _agg2_call(HS1.reshape(2 * NPAD, D), srcBC, dst16)
    HS2 = _mid_call(y1, HS1, dis, W[:, 1], b[:, 1][:, None, :])
    y2 = _agg2_call(HS2.reshape(2 * NPAD, D), srcBC, dst16)
    ln = _fin_call(y2, HS2, dis, W[:, 2], b[:, 2][:, None, :],
                   gamma[None], beta[None])
    batchs = jnp.ones((_SZ_C, batch.shape[0]), dtype=x.dtype) * batch.astype(x.dtype)
    return (ln, batchs)


# speedup vs baseline: 23.6589x; 1.3260x over previous
import functools

import jax
import jax.numpy as jnp
from jax import lax
from jax.experimental import pallas as pl
from jax.experimental.pallas import tpu as pltpu
from jax.experimental.pallas import tpu_sc as plsc

N = 10000
NPAD = 10112
D = 128
E = 320000
EB = 128
NBLK1 = 80
NBLK2 = 160
G = 16
EPAD = 32 * NBLK1 * EB
ACCN = 10240
_SZ_C = 2
_SZ_L = 3


def _sc_mesh():
    return plsc.VectorSubcoreMesh(core_axis_name="c", subcore_axis_name="s")


def _make_deg_kernel():
    @functools.partial(
        pl.kernel,
        out_type=jax.ShapeDtypeStruct((2, ACCN), jnp.float32),
        mesh=_sc_mesh(),
        scratch_types=[
            pltpu.VMEM((NBLK1, EB), jnp.int32),
            pltpu.VMEM((EB,), jnp.float32),
            pltpu.VMEM((640,), jnp.float32),
            pltpu.VMEM_SHARED((ACCN,), jnp.float32),
        ],
    )
    def deg_kernel(dst_hbm, degp_hbm, dst_v, ones_v, z_v, acc):
        c = lax.axis_index("c")
        s = lax.axis_index("s")
        w = c * 16 + s
        for j in range(EB // 16):
            ones_v[pl.ds(j * 16, 16)] = jnp.ones((16,), jnp.float32)
        for j in range(640 // 16):
            z_v[pl.ds(j * 16, 16)] = jnp.zeros((16,), jnp.float32)
        pltpu.sync_copy(z_v, acc.at[pl.ds(s * 640, 640)])
        pltpu.sync_copy(dst_hbm.at[w], dst_v)
        plsc.subcore_barrier()

        def body(i, carry):
            pltpu.sync_copy(ones_v, acc.at[dst_v.at[i]], add=True)
            return carry

        lax.fori_loop(0, NBLK1, body, 0)
        plsc.subcore_barrier()
        pltpu.sync_copy(acc.at[pl.ds(s * 640, 640)],
                        degp_hbm.at[c, pl.ds(s * 640, 640)])

    return deg_kernel


def _make_agg_kernel(two_ch):
    nblk = NBLK2 if two_ch else NBLK1
    chunk = nblk * EB

    @functools.partial(
        pl.kernel,
        out_type=jax.ShapeDtypeStruct((2, NPAD, D), jnp.float32),
        mesh=_sc_mesh(),
        scratch_types=[
            pltpu.VMEM((G * EB,), jnp.int32),
            pltpu.VMEM((G * EB,), jnp.int32),
            pltpu.VMEM((G, EB), jnp.int32),
            pltpu.VMEM((G, EB), jnp.int32),
            pltpu.VMEM((2, EB, D), jnp.float32),
            pltpu.VMEM_SHARED((NPAD, D), jnp.float32),
            pltpu.SemaphoreType.DMA,
            pltpu.SemaphoreType.DMA,
            pltpu.SemaphoreType.DMA,
            pltpu.SemaphoreType.DMA,
        ],
    )
    def agg_kernel(tbl_hbm, src_hbm, dst_hbm, out_hbm, src_g0, src_g1,
                   dst_g0, dst_g1, rows_v, acc, gsem0, gsem1, ssem, isem):
        c = lax.axis_index("c")
        s = lax.axis_index("s")
        w = c * 16 + s
        ngrp = nblk // G
        gsems = (gsem0, gsem1)
        src_gs = (src_g0, src_g1)
        dst_gs = (dst_g0, dst_g1)

        def zbody(i, carry):
            for j in range(D // 16):
                rows_v[0, i, pl.ds(j * 16, 16)] = jnp.zeros((16,), jnp.float32)
            return carry

        lax.fori_loop(0, EB, zbody, 0)
        base = s * 632
        z0 = rows_v.at[0]
        for k in range(4):
            pltpu.sync_copy(z0, acc.at[pl.ds(base + k * EB, EB)])
        pltpu.sync_copy(z0.at[pl.ds(0, 120)], acc.at[pl.ds(base + 4 * EB, 120)])
        plsc.subcore_barrier()

        def _src_ref(g):
            if two_ch:
                return src_hbm.at[c, pl.ds(s * chunk + g * (G * EB), G * EB)]
            return src_hbm.at[pl.ds(w * chunk + g * (G * EB), G * EB)]

        def _dst_ref(g):
            if two_ch:
                return dst_hbm.at[s, pl.ds(g * G, G)]
            return dst_hbm.at[w, pl.ds(g * G, G)]

        def _idx_start(g, slot):
            pltpu.async_copy(_src_ref(g), src_gs[slot], isem)
            pltpu.async_copy(_dst_ref(g), dst_gs[slot], isem)

        def _idx_wait(g, slot):
            pltpu.make_async_copy(_src_ref(g), src_gs[slot], isem).wait()
            pltpu.make_async_copy(_dst_ref(g), dst_gs[slot], isem).wait()

        def _gref(gslot, j, p):
            return (tbl_hbm.at[src_gs[gslot].at[pl.ds(j * EB, EB)]],
                    rows_v.at[p], gsems[p])

        _idx_start(0, 0)
        for gi in range(ngrp):
            gslot = gi % 2
            _idx_wait(gi, gslot)
            if gi + 1 < ngrp:
                _idx_start(gi + 1, (gi + 1) % 2)
            pltpu.async_copy(*_gref(gslot, 0, 0))
            for j in range(G):
                p = j % 2
                pltpu.make_async_copy(*_gref(gslot, j, p)).wait()
                if j + 1 < G:
                    pltpu.async_copy(*_gref(gslot, j + 1, (j + 1) % 2))
                pltpu.async_copy(rows_v.at[p], acc.at[dst_gs[gslot].at[j]],
                                 ssem, add=True).wait()
        plsc.subcore_barrier()
        dbase = s * 632
        pltpu.sync_copy(acc.at[pl.ds(dbase, 632)],
                        out_hbm.at[c, pl.ds(dbase, 632)])

    return agg_kernel


_deg_call = _make_deg_kernel()
_agg1_call = _make_agg_kernel(False)
_agg2_call = _make_agg_kernel(True)


_BRD = 2528
_BRL = 2528
_BRF = 2000


def _dis_body(degp_ref, x_ref, dis_ref, xs_ref):
    deg = degp_ref[:, 0] + degp_ref[:, 1] + 1.0
    dis = lax.rsqrt(deg)[:, None]
    dis_ref[...] = dis
    xs_ref[...] = dis * x_ref[...]


def _dis_call(degp, xp):
    grid = (NPAD // _BRD,)
    return pl.pallas_call(
        _dis_body,
        grid=grid,
        in_specs=[
            pl.BlockSpec((_BRD, 2), lambda i: (i, 0)),
            pl.BlockSpec((_BRD, D), lambda i: (i, 0)),
        ],
        out_specs=[
            pl.BlockSpec((_BRD, 1), lambda i: (i, 0)),
            pl.BlockSpec((_BRD, D), lambda i: (i, 0)),
        ],
        out_shape=[
            jax.ShapeDtypeStruct((NPAD, 1), jnp.float32),
            jax.ShapeDtypeStruct((NPAD, D), jnp.float32),
        ],
    )(degp, xp)


def _l0_body(y_ref, xs_ref, dis_ref, W_ref, b_ref, o_ref):
    dis = dis_ref[...]
    u = dis * (y_ref[0] + y_ref[1] + xs_ref[...])
    for c in range(_SZ_C):
        g = jnp.dot(u, W_ref[c], preferred_element_type=jnp.float32) + b_ref[c]
        h = jnp.maximum(g, 0.0) + g
        o_ref[c] = dis * h


def _l0_call(y, xs, dis, Wl, bl):
    grid = (NPAD // _BRL,)
    return pl.pallas_call(
        _l0_body,
        grid=grid,
        in_specs=[
            pl.BlockSpec((2, _BRL, D), lambda i: (0, i, 0)),
            pl.BlockSpec((_BRL, D), lambda i: (i, 0)),
            pl.BlockSpec((_BRL, 1), lambda i: (i, 0)),
            pl.BlockSpec((2, D, D), lambda i: (0, 0, 0)),
            pl.BlockSpec((2, 1, D), lambda i: (0, 0, 0)),
        ],
        out_specs=pl.BlockSpec((2, _BRL, D), lambda i: (0, i, 0)),
        out_shape=jax.ShapeDtypeStruct((2, NPAD, D), jnp.float32),
    )(y, xs, dis, Wl, bl)


def _mid_body(y_ref, hs_ref, dis_ref, W_ref, b_ref, o_ref):
    dis = dis_ref[...]
    for c in range(_SZ_C):
        v = dis * (y_ref[c] + hs_ref[c])
        g = jnp.dot(v, W_ref[c], preferred_element_type=jnp.float32) + b_ref[c]
        h = jnp.maximum(g, 0.0) + g
        o_ref[c] = dis * h


def _mid_call(y, hs, dis, Wl, bl):
    grid = (NPAD // _BRL,)
    return pl.pallas_call(
        _mid_body,
        grid=grid,
        in_specs=[
            pl.BlockSpec((2, _BRL, D), lambda i: (0, i, 0)),
            pl.BlockSpec((2, _BRL, D), lambda i: (0, i, 0)),
            pl.BlockSpec((_BRL, 1), lambda i: (i, 0)),
            pl.BlockSpec((2, D, D), lambda i: (0, 0, 0)),
            pl.BlockSpec((2, 1, D), lambda i: (0, 0, 0)),
        ],
        out_specs=pl.BlockSpec((2, _BRL, D), lambda i: (0, i, 0)),
        out_shape=jax.ShapeDtypeStruct((2, NPAD, D), jnp.float32),
    )(y, hs, dis, Wl, bl)


def _fin_body(y_ref, hs_ref, dis_ref, W_ref, b_ref, gamma_ref, beta_ref, o_ref):
    dis = dis_ref[...]
    for c in range(_SZ_C):
        v = dis * (y_ref[c] + hs_ref[c])
        g = jnp.dot(v, W_ref[c], preferred_element_type=jnp.float32) + b_ref[c]
        h = jnp.maximum(g, 0.0) + g
        mu = jnp.mean(h, axis=-1, keepdims=True)
        var = jnp.mean((h - mu) ** 2, axis=-1, keepdims=True)
        o_ref[c] = (h - mu) * lax.rsqrt(var + 1e-6) * gamma_ref[...] + beta_ref[...]


def _fin_call(y, hs, dis, Wl, bl, gamma, beta):
    grid = (N // _BRF,)
    return pl.pallas_call(
        _fin_body,
        grid=grid,
        in_specs=[
            pl.BlockSpec((2, _BRF, D), lambda i: (0, i, 0)),
            pl.BlockSpec((2, _BRF, D), lambda i: (0, i, 0)),
            pl.BlockSpec((_BRF, 1), lambda i: (i, 0)),
            pl.BlockSpec((2, D, D), lambda i: (0, 0, 0)),
            pl.BlockSpec((2, 1, D), lambda i: (0, 0, 0)),
            pl.BlockSpec((1, D), lambda i: (0, 0)),
            pl.BlockSpec((1, D), lambda i: (0, 0)),
        ],
        out_specs=pl.BlockSpec((2, _BRF, D), lambda i: (0, i, 0)),
        out_shape=jax.ShapeDtypeStruct((2, N, D), jnp.float32),
    )(y, hs, dis, Wl, bl, gamma, beta)


def kernel(x, edge, batch, W, b, gamma, beta):
    src, dst = edge[0], edge[1]
    npad_e = EPAD - E
    ar = jnp.arange(npad_e, dtype=jnp.int32)
    pad_src = (ar * 997) % N
    pad_dst = N + (ar % (NPAD - N))
    srcA = jnp.concatenate([src, pad_src])
    dstA = jnp.concatenate([dst, pad_dst])
    dst32 = dstA.reshape(32, NBLK1, EB)
    dst16 = dstA.reshape(16, NBLK2, EB)
    srcBC = jnp.stack([srcA, srcA + NPAD])
    xp = jnp.pad(x, ((0, NPAD - N), (0, 0)))

    degp = _deg_call(dst32)
    dis, xs = _dis_call(degp.T[:NPAD], xp)

    yA = _agg1_call(xs, srcA, dst32)
    HS1 = _l0_call(yA, xs, dis, W[:, 0], b[:, 0][:, None, :])
    y1 = _agg2_call(HS1.reshape(2 * NPAD, D), srcBC, dst16)
    HS2 = _mid_call(y1, HS1, dis, W[:, 1], b[:, 1][:, None, :])
    y2 = _agg2_call(HS2.reshape(2 * NPAD, D), srcBC, dst16)
    ln = _fin_call(y2, HS2, dis, W[:, 2], b[:, 2][:, None, :],
                   gamma[None], beta[None])
    batchs = jnp.ones((_SZ_C, batch.shape[0]), dtype=x.dtype) * batch.astype(x.dtype)
    return (ln, batchs)


# speedup vs baseline: 24.1871x; 1.0223x over previous
import functools

import jax
import jax.numpy as jnp
from jax import lax
from jax.experimental import pallas as pl
from jax.experimental.pallas import tpu as pltpu
from jax.experimental.pallas import tpu_sc as plsc

N = 10000
NPAD = 10112
D = 128
E = 320000
EB = 128
NBLK1 = 80
NBLK2 = 160
G = 16
EPAD = 32 * NBLK1 * EB
ACCN = 10240
_SZ_C = 2
_SZ_L = 3


def _sc_mesh():
    return plsc.VectorSubcoreMesh(core_axis_name="c", subcore_axis_name="s")


def _make_deg_kernel():
    @functools.partial(
        pl.kernel,
        out_type=jax.ShapeDtypeStruct((2, ACCN), jnp.float32),
        mesh=_sc_mesh(),
        scratch_types=[
            pltpu.VMEM((NBLK1, EB), jnp.int32),
            pltpu.VMEM((EB,), jnp.float32),
            pltpu.VMEM((640,), jnp.float32),
            pltpu.VMEM_SHARED((ACCN,), jnp.float32),
        ],
    )
    def deg_kernel(dst_hbm, degp_hbm, dst_v, ones_v, z_v, acc):
        c = lax.axis_index("c")
        s = lax.axis_index("s")
        w = c * 16 + s
        for j in range(EB // 16):
            ones_v[pl.ds(j * 16, 16)] = jnp.ones((16,), jnp.float32)
        for j in range(640 // 16):
            z_v[pl.ds(j * 16, 16)] = jnp.zeros((16,), jnp.float32)
        pltpu.sync_copy(z_v, acc.at[pl.ds(s * 640, 640)])
        pltpu.sync_copy(dst_hbm.at[w], dst_v)
        plsc.subcore_barrier()

        def body(i, carry):
            pltpu.sync_copy(ones_v, acc.at[dst_v.at[i]], add=True)
            return carry

        lax.fori_loop(0, NBLK1, body, 0)
        plsc.subcore_barrier()
        pltpu.sync_copy(acc.at[pl.ds(s * 640, 640)],
                        degp_hbm.at[c, pl.ds(s * 640, 640)])

    return deg_kernel


def _make_agg_kernel(two_ch):
    nblk = NBLK2 if two_ch else NBLK1
    chunk = nblk * EB

    @functools.partial(
        pl.kernel,
        out_type=jax.ShapeDtypeStruct((2, NPAD, D), jnp.float32),
        mesh=_sc_mesh(),
        scratch_types=[
            pltpu.VMEM((G * EB,), jnp.int32),
            pltpu.VMEM((G * EB,), jnp.int32),
            pltpu.VMEM((G, EB), jnp.int32),
            pltpu.VMEM((G, EB), jnp.int32),
            pltpu.VMEM((2, EB, D), jnp.float32),
            pltpu.VMEM_SHARED((NPAD, D), jnp.float32),
            pltpu.SemaphoreType.DMA,
            pltpu.SemaphoreType.DMA,
            pltpu.SemaphoreType.DMA,
            pltpu.SemaphoreType.DMA,
        ],
    )
    def agg_kernel(tbl_hbm, src_hbm, dst_hbm, out_hbm, src_g0, src_g1,
                   dst_g0, dst_g1, rows_v, acc, gsem0, gsem1, ssem, isem):
        c = lax.axis_index("c")
        s = lax.axis_index("s")
        w = c * 16 + s
        ngrp = nblk // G
        gsems = (gsem0, gsem1)
        src_gs = (src_g0, src_g1)
        dst_gs = (dst_g0, dst_g1)

        def zbody(i, carry):
            for j in range(D // 16):
                rows_v[0, i, pl.ds(j * 16, 16)] = jnp.zeros((16,), jnp.float32)
            return carry

        lax.fori_loop(0, EB, zbody, 0)
        base = s * 632
        z0 = rows_v.at[0]
        for k in range(4):
            pltpu.sync_copy(z0, acc.at[pl.ds(base + k * EB, EB)])
        pltpu.sync_copy(z0.at[pl.ds(0, 120)], acc.at[pl.ds(base + 4 * EB, 120)])
        plsc.subcore_barrier()

        def _src_ref(g):
            if two_ch:
                return src_hbm.at[c, pl.ds(s * chunk + g * (G * EB), G * EB)]
            return src_hbm.at[pl.ds(w * chunk + g * (G * EB), G * EB)]

        def _dst_ref(g):
            if two_ch:
                return dst_hbm.at[s, pl.ds(g * G, G)]
            return dst_hbm.at[w, pl.ds(g * G, G)]

        def _idx_start(g, slot):
            pltpu.async_copy(_src_ref(g), src_gs[slot], isem)
            pltpu.async_copy(_dst_ref(g), dst_gs[slot], isem)

        def _idx_wait(g, slot):
            pltpu.make_async_copy(_src_ref(g), src_gs[slot], isem).wait()
            pltpu.make_async_copy(_dst_ref(g), dst_gs[slot], isem).wait()

        def _gref(gslot, j, p):
            return (tbl_hbm.at[src_gs[gslot].at[pl.ds(j * EB, EB)]],
                    rows_v.at[p], gsems[p])

        def _gblk(jj):
            return ((jj // G) % 2, jj % G, jj % 2)

        _idx_start(0, 0)
        _idx_wait(0, 0)
        gs0, j0, p0 = _gblk(0)
        pltpu.async_copy(*_gref(gs0, j0, p0))
        for jj in range(nblk):
            gslot, j, p = _gblk(jj)
            gi = jj // G
            if j == 0 and gi + 1 < ngrp:
                _idx_start(gi + 1, (gi + 1) % 2)
            pltpu.make_async_copy(*_gref(gslot, j, p)).wait()
            nxt = jj + 1
            if nxt < nblk:
                ngs, nj, np_ = _gblk(nxt)
                if nj == 0:
                    _idx_wait(nxt // G, ngs)
                pltpu.async_copy(*_gref(ngs, nj, np_))
            pltpu.async_copy(rows_v.at[p], acc.at[dst_gs[gslot].at[j]],
                             ssem, add=True).wait()
        plsc.subcore_barrier()
        dbase = s * 632
        pltpu.sync_copy(acc.at[pl.ds(dbase, 632)],
                        out_hbm.at[c, pl.ds(dbase, 632)])

    return agg_kernel


_deg_call = _make_deg_kernel()
_agg1_call = _make_agg_kernel(False)
_agg2_call = _make_agg_kernel(True)


_BRD = 2528
_BRL = 2528
_BRF = 2000


def _dis_body(degp_ref, x_ref, dis_ref, xs_ref):
    deg = degp_ref[:, 0] + degp_ref[:, 1] + 1.0
    dis = lax.rsqrt(deg)[:, None]
    dis_ref[...] = dis
    xs_ref[...] = dis * x_ref[...]


def _dis_call(degp, xp):
    grid = (NPAD // _BRD,)
    return pl.pallas_call(
        _dis_body,
        grid=grid,
        in_specs=[
            pl.BlockSpec((_BRD, 2), lambda i: (i, 0)),
            pl.BlockSpec((_BRD, D), lambda i: (i, 0)),
        ],
        out_specs=[
            pl.BlockSpec((_BRD, 1), lambda i: (i, 0)),
            pl.BlockSpec((_BRD, D), lambda i: (i, 0)),
        ],
        out_shape=[
            jax.ShapeDtypeStruct((NPAD, 1), jnp.float32),
            jax.ShapeDtypeStruct((NPAD, D), jnp.float32),
        ],
    )(degp, xp)


def _l0_body(y_ref, xs_ref, dis_ref, W_ref, b_ref, o_ref):
    dis = dis_ref[...]
    u = dis * (y_ref[0] + y_ref[1] + xs_ref[...])
    for c in range(_SZ_C):
        g = jnp.dot(u, W_ref[c], preferred_element_type=jnp.float32) + b_ref[c]
        h = jnp.maximum(g, 0.0) + g
        o_ref[c] = dis * h


def _l0_call(y, xs, dis, Wl, bl):
    grid = (NPAD // _BRL,)
    return pl.pallas_call(
        _l0_body,
        grid=grid,
        in_specs=[
            pl.BlockSpec((2, _BRL, D), lambda i: (0, i, 0)),
            pl.BlockSpec((_BRL, D), lambda i: (i, 0)),
            pl.BlockSpec((_BRL, 1), lambda i: (i, 0)),
            pl.BlockSpec((2, D, D), lambda i: (0, 0, 0)),
            pl.BlockSpec((2, 1, D), lambda i: (0, 0, 0)),
        ],
        out_specs=pl.BlockSpec((2, _BRL, D), lambda i: (0, i, 0)),
        out_shape=jax.ShapeDtypeStruct((2, NPAD, D), jnp.float32),
    )(y, xs, dis, Wl, bl)


def _mid_body(y_ref, hs_ref, dis_ref, W_ref, b_ref, o_ref):
    dis = dis_ref[...]
    for c in range(_SZ_C):
        v = dis * (y_ref[c] + hs_ref[c])
        g = jnp.dot(v, W_ref[c], preferred_element_type=jnp.float32) + b_ref[c]
        h = jnp.maximum(g, 0.0) + g
        o_ref[c] = dis * h


def _mid_call(y, hs, dis, Wl, bl):
    grid = (NPAD // _BRL,)
    return pl.pallas_call(
        _mid_body,
        grid=grid,
        in_specs=[
            pl.BlockSpec((2, _BRL, D), lambda i: (0, i, 0)),
            pl.BlockSpec((2, _BRL, D), lambda i: (0, i, 0)),
            pl.BlockSpec((_BRL, 1), lambda i: (i, 0)),
            pl.BlockSpec((2, D, D), lambda i: (0, 0, 0)),
            pl.BlockSpec((2, 1, D), lambda i: (0, 0, 0)),
        ],
        out_specs=pl.BlockSpec((2, _BRL, D), lambda i: (0, i, 0)),
        out_shape=jax.ShapeDtypeStruct((2, NPAD, D), jnp.float32),
    )(y, hs, dis, Wl, bl)


def _fin_body(y_ref, hs_ref, dis_ref, W_ref, b_ref, gamma_ref, beta_ref, o_ref):
    dis = dis_ref[...]
    for c in range(_SZ_C):
        v = dis * (y_ref[c] + hs_ref[c])
        g = jnp.dot(v, W_ref[c], preferred_element_type=jnp.float32) + b_ref[c]
        h = jnp.maximum(g, 0.0) + g
        mu = jnp.mean(h, axis=-1, keepdims=True)
        var = jnp.mean((h - mu) ** 2, axis=-1, keepdims=True)
        o_ref[c] = (h - mu) * lax.rsqrt(var + 1e-6) * gamma_ref[...] + beta_ref[...]


def _fin_call(y, hs, dis, Wl, bl, gamma, beta):
    grid = (N // _BRF,)
    return pl.pallas_call(
        _fin_body,
        grid=grid,
        in_specs=[
            pl.BlockSpec((2, _BRF, D), lambda i: (0, i, 0)),
            pl.BlockSpec((2, _BRF, D), lambda i: (0, i, 0)),
            pl.BlockSpec((_BRF, 1), lambda i: (i, 0)),
            pl.BlockSpec((2, D, D), lambda i: (0, 0, 0)),
            pl.BlockSpec((2, 1, D), lambda i: (0, 0, 0)),
            pl.BlockSpec((1, D), lambda i: (0, 0)),
            pl.BlockSpec((1, D), lambda i: (0, 0)),
        ],
        out_specs=pl.BlockSpec((2, _BRF, D), lambda i: (0, i, 0)),
        out_shape=jax.ShapeDtypeStruct((2, N, D), jnp.float32),
    )(y, hs, dis, Wl, bl, gamma, beta)


def kernel(x, edge, batch, W, b, gamma, beta):
    src, dst = edge[0], edge[1]
    npad_e = EPAD - E
    ar = jnp.arange(npad_e, dtype=jnp.int32)
    pad_src = (ar * 997) % N
    pad_dst = N + (ar % (NPAD - N))
    srcA = jnp.concatenate([src, pad_src])
    dstA = jnp.concatenate([dst, pad_dst])
    dst32 = dstA.reshape(32, NBLK1, EB)
    dst16 = dstA.reshape(16, NBLK2, EB)
    srcBC = jnp.stack([srcA, srcA + NPAD])
    xp = jnp.pad(x, ((0, NPAD - N), (0, 0)))

    degp = _deg_call(dst32)
    dis, xs = _dis_call(degp.T[:NPAD], xp)

    yA = _agg1_call(xs, srcA, dst32)
    HS1 = _l0_call(yA, xs, dis, W[:, 0], b[:, 0][:, None, :])
    y1 = _agg2_call(HS1.reshape(2 * NPAD, D), srcBC, dst16)
    HS2 = _mid_call(y1, HS1, dis, W[:, 1], b[:, 1][:, None, :])
    y2 = _agg2_call(HS2.reshape(2 * NPAD, D), srcBC, dst16)
    ln = _fin_call(y2, HS2, dis, W[:, 2], b[:, 2][:, None, :],
                   gamma[None], beta[None])
    batchs = jnp.ones((_SZ_C, batch.shape[0]), dtype=x.dtype) * batch.astype(x.dtype)
    return (ln, batchs)


# speedup vs baseline: 27.8732x; 1.1524x over previous
import functools

import jax
import jax.numpy as jnp
from jax import lax
from jax.experimental import pallas as pl
from jax.experimental.pallas import tpu as pltpu
from jax.experimental.pallas import tpu_sc as plsc

N = 10000
NPAD = 10112
D = 128
E = 320000
EB = 64
NBLK1 = 160
NBLK2 = 320
G = 32
EPAD = 32 * NBLK1 * EB
ACCN = 10240
_SZ_C = 2
_SZ_L = 3


def _sc_mesh():
    return plsc.VectorSubcoreMesh(core_axis_name="c", subcore_axis_name="s")


def _make_deg_kernel():
    @functools.partial(
        pl.kernel,
        out_type=jax.ShapeDtypeStruct((2, ACCN), jnp.float32),
        mesh=_sc_mesh(),
        scratch_types=[
            pltpu.VMEM((NBLK1, EB), jnp.int32),
            pltpu.VMEM((EB,), jnp.float32),
            pltpu.VMEM((640,), jnp.float32),
            pltpu.VMEM_SHARED((ACCN,), jnp.float32),
        ],
    )
    def deg_kernel(dst_hbm, degp_hbm, dst_v, ones_v, z_v, acc):
        c = lax.axis_index("c")
        s = lax.axis_index("s")
        w = c * 16 + s
        for j in range(EB // 16):
            ones_v[pl.ds(j * 16, 16)] = jnp.ones((16,), jnp.float32)
        for j in range(640 // 16):
            z_v[pl.ds(j * 16, 16)] = jnp.zeros((16,), jnp.float32)
        pltpu.sync_copy(z_v, acc.at[pl.ds(s * 640, 640)])
        pltpu.sync_copy(dst_hbm.at[w], dst_v)
        plsc.subcore_barrier()

        def body(i, carry):
            pltpu.sync_copy(ones_v, acc.at[dst_v.at[i]], add=True)
            return carry

        lax.fori_loop(0, NBLK1, body, 0)
        plsc.subcore_barrier()
        pltpu.sync_copy(acc.at[pl.ds(s * 640, 640)],
                        degp_hbm.at[c, pl.ds(s * 640, 640)])

    return deg_kernel


def _make_agg_kernel(two_ch):
    nblk = NBLK2 if two_ch else NBLK1
    chunk = nblk * EB

    @functools.partial(
        pl.kernel,
        out_type=jax.ShapeDtypeStruct((2, NPAD, D), jnp.float32),
        mesh=_sc_mesh(),
        scratch_types=[
            pltpu.VMEM((G * EB,), jnp.int32),
            pltpu.VMEM((G * EB,), jnp.int32),
            pltpu.VMEM((G, EB), jnp.int32),
            pltpu.VMEM((G, EB), jnp.int32),
            pltpu.VMEM((4, EB, D), jnp.float32),
            pltpu.VMEM_SHARED((NPAD, D), jnp.float32),
            pltpu.SemaphoreType.DMA,
            pltpu.SemaphoreType.DMA,
            pltpu.SemaphoreType.DMA,
            pltpu.SemaphoreType.DMA,
            pltpu.SemaphoreType.DMA,
            pltpu.SemaphoreType.DMA,
            pltpu.SemaphoreType.DMA,
            pltpu.SemaphoreType.DMA,
            pltpu.SemaphoreType.DMA,
        ],
    )
    def agg_kernel(tbl_hbm, src_hbm, dst_hbm, out_hbm, src_g0, src_g1,
                   dst_g0, dst_g1, rows_v, acc, gsem0, gsem1, gsem2, gsem3,
                   ssem0, ssem1, ssem2, ssem3, isem):
        c = lax.axis_index("c")
        s = lax.axis_index("s")
        w = c * 16 + s
        ngrp = nblk // G
        gsems = (gsem0, gsem1, gsem2, gsem3)
        ssems = (ssem0, ssem1, ssem2, ssem3)
        src_gs = (src_g0, src_g1)
        dst_gs = (dst_g0, dst_g1)

        def zbody(i, carry):
            for j in range(D // 16):
                rows_v[0, i, pl.ds(j * 16, 16)] = jnp.zeros((16,), jnp.float32)
            return carry

        lax.fori_loop(0, EB, zbody, 0)
        base = s * 632
        z0 = rows_v.at[0]
        for k in range(9):
            pltpu.sync_copy(z0, acc.at[pl.ds(base + k * EB, EB)])
        pltpu.sync_copy(z0.at[pl.ds(0, 56)], acc.at[pl.ds(base + 9 * EB, 56)])
        plsc.subcore_barrier()

        def _src_ref(g):
            if two_ch:
                return src_hbm.at[c, pl.ds(s * chunk + g * (G * EB), G * EB)]
            return src_hbm.at[pl.ds(w * chunk + g * (G * EB), G * EB)]

        def _dst_ref(g):
            if two_ch:
                return dst_hbm.at[s, pl.ds(g * G, G)]
            return dst_hbm.at[w, pl.ds(g * G, G)]

        def _idx_start(g, slot):
            pltpu.async_copy(_src_ref(g), src_gs[slot], isem)
            pltpu.async_copy(_dst_ref(g), dst_gs[slot], isem)

        def _idx_wait(g, slot):
            pltpu.make_async_copy(_src_ref(g), src_gs[slot], isem).wait()
            pltpu.make_async_copy(_dst_ref(g), dst_gs[slot], isem).wait()

        def _gref(slot, j, p):
            return (tbl_hbm.at[src_gs[slot].at[pl.ds(j * EB, EB)]],
                    rows_v.at[p], gsems[p])

        def _sref(slot, j, p):
            return (rows_v.at[p], acc.at[dst_gs[slot].at[j]], ssems[p])

        def run_group(slot):
            for k in range(3):
                pltpu.async_copy(*_gref(slot, k, k))
            for j in range(G):
                p = j % 4
                pltpu.make_async_copy(*_gref(slot, j, p)).wait()
                pltpu.async_copy(*_sref(slot, j, p), add=True)
                if j >= 1:
                    pltpu.make_async_copy(*_sref(slot, j - 1, (j - 1) % 4)).wait()
                if j + 3 < G:
                    pltpu.async_copy(*_gref(slot, j + 3, (j + 3) % 4))
            pltpu.make_async_copy(*_sref(slot, G - 1, (G - 1) % 4)).wait()

        npair = ngrp // 2
        _idx_start(0, 0)

        def pair(gi2, carry):
            gA = 2 * gi2
            _idx_wait(gA, 0)
            _idx_start(gA + 1, 1)
            run_group(0)
            _idx_wait(gA + 1, 1)
            _idx_start(jnp.minimum(gA + 2, ngrp - 1), 0)
            run_group(1)
            return carry

        lax.fori_loop(0, npair, pair, 0)
        if ngrp % 2 == 1:
            _idx_wait(ngrp - 1, 0)
            run_group(0)
        else:
            _idx_wait(ngrp - 1, 0)
        plsc.subcore_barrier()
        dbase = s * 632
        pltpu.sync_copy(acc.at[pl.ds(dbase, 632)],
                        out_hbm.at[c, pl.ds(dbase, 632)])

    return agg_kernel


_deg_call = _make_deg_kernel()
_agg1_call = _make_agg_kernel(False)
_agg2_call = _make_agg_kernel(True)


_BRD = 2528
_BRL = 2528
_BRF = 2000


def _dis_body(degp_ref, x_ref, dis_ref, xs_ref):
    deg = degp_ref[:, 0] + degp_ref[:, 1] + 1.0
    dis = lax.rsqrt(deg)[:, None]
    dis_ref[...] = dis
    xs_ref[...] = dis * x_ref[...]


def _dis_call(degp, xp):
    grid = (NPAD // _BRD,)
    return pl.pallas_call(
        _dis_body,
        grid=grid,
        in_specs=[
            pl.BlockSpec((_BRD, 2), lambda i: (i, 0)),
            pl.BlockSpec((_BRD, D), lambda i: (i, 0)),
        ],
        out_specs=[
            pl.BlockSpec((_BRD, 1), lambda i: (i, 0)),
            pl.BlockSpec((_BRD, D), lambda i: (i, 0)),
        ],
        out_shape=[
            jax.ShapeDtypeStruct((NPAD, 1), jnp.float32),
            jax.ShapeDtypeStruct((NPAD, D), jnp.float32),
        ],
    )(degp, xp)


def _l0_body(y_ref, xs_ref, dis_ref, W_ref, b_ref, o_ref):
    dis = dis_ref[...]
    u = dis * (y_ref[0] + y_ref[1] + xs_ref[...])
    for c in range(_SZ_C):
        g = jnp.dot(u, W_ref[c], preferred_element_type=jnp.float32) + b_ref[c]
        h = jnp.maximum(g, 0.0) + g
        o_ref[c] = dis * h


def _l0_call(y, xs, dis, Wl, bl):
    grid = (NPAD // _BRL,)
    return pl.pallas_call(
        _l0_body,
        grid=grid,
        in_specs=[
            pl.BlockSpec((2, _BRL, D), lambda i: (0, i, 0)),
            pl.BlockSpec((_BRL, D), lambda i: (i, 0)),
            pl.BlockSpec((_BRL, 1), lambda i: (i, 0)),
            pl.BlockSpec((2, D, D), lambda i: (0, 0, 0)),
            pl.BlockSpec((2, 1, D), lambda i: (0, 0, 0)),
        ],
        out_specs=pl.BlockSpec((2, _BRL, D), lambda i: (0, i, 0)),
        out_shape=jax.ShapeDtypeStruct((2, NPAD, D), jnp.float32),
    )(y, xs, dis, Wl, bl)


def _mid_body(y_ref, hs_ref, dis_ref, W_ref, b_ref, o_ref):
    dis = dis_ref[...]
    for c in range(_SZ_C):
        v = dis * (y_ref[c] + hs_ref[c])
        g = jnp.dot(v, W_ref[c], preferred_element_type=jnp.float32) + b_ref[c]
        h = jnp.maximum(g, 0.0) + g
        o_ref[c] = dis * h


def _mid_call(y, hs, dis, Wl, bl):
    grid = (NPAD // _BRL,)
    return pl.pallas_call(
        _mid_body,
        grid=grid,
        in_specs=[
            pl.BlockSpec((2, _BRL, D), lambda i: (0, i, 0)),
            pl.BlockSpec((2, _BRL, D), lambda i: (0, i, 0)),
            pl.BlockSpec((_BRL, 1), lambda i: (i, 0)),
            pl.BlockSpec((2, D, D), lambda i: (0, 0, 0)),
            pl.BlockSpec((2, 1, D), lambda i: (0, 0, 0)),
        ],
        out_specs=pl.BlockSpec((2, _BRL, D), lambda i: (0, i, 0)),
        out_shape=jax.ShapeDtypeStruct((2, NPAD, D), jnp.float32),
    )(y, hs, dis, Wl, bl)


def _fin_body(y_ref, hs_ref, dis_ref, W_ref, b_ref, gamma_ref, beta_ref, o_ref):
    dis = dis_ref[...]
    for c in range(_SZ_C):
        v = dis * (y_ref[c] + hs_ref[c])
        g = jnp.dot(v, W_ref[c], preferred_element_type=jnp.float32) + b_ref[c]
        h = jnp.maximum(g, 0.0) + g
        mu = jnp.mean(h, axis=-1, keepdims=True)
        var = jnp.mean((h - mu) ** 2, axis=-1, keepdims=True)
        o_ref[c] = (h - mu) * lax.rsqrt(var + 1e-6) * gamma_ref[...] + beta_ref[...]


def _fin_call(y, hs, dis, Wl, bl, gamma, beta):
    grid = (N // _BRF,)
    return pl.pallas_call(
        _fin_body,
        grid=grid,
        in_specs=[
            pl.BlockSpec((2, _BRF, D), lambda i: (0, i, 0)),
            pl.BlockSpec((2, _BRF, D), lambda i: (0, i, 0)),
            pl.BlockSpec((_BRF, 1), lambda i: (i, 0)),
            pl.BlockSpec((2, D, D), lambda i: (0, 0, 0)),
            pl.BlockSpec((2, 1, D), lambda i: (0, 0, 0)),
            pl.BlockSpec((1, D), lambda i: (0, 0)),
            pl.BlockSpec((1, D), lambda i: (0, 0)),
        ],
        out_specs=pl.BlockSpec((2, _BRF, D), lambda i: (0, i, 0)),
        out_shape=jax.ShapeDtypeStruct((2, N, D), jnp.float32),
    )(y, hs, dis, Wl, bl, gamma, beta)


def kernel(x, edge, batch, W, b, gamma, beta):
    src, dst = edge[0], edge[1]
    npad_e = EPAD - E
    ar = jnp.arange(npad_e, dtype=jnp.int32)
    pad_src = (ar * 997) % N
    pad_dst = N + (ar % (NPAD - N))
    srcA = jnp.concatenate([src, pad_src])
    dstA = jnp.concatenate([dst, pad_dst])
    dst32 = dstA.reshape(32, NBLK1, EB)
    dst16 = dstA.reshape(16, NBLK2, EB)
    srcBC = jnp.stack([srcA, srcA + NPAD])
    xp = jnp.pad(x, ((0, NPAD - N), (0, 0)))

    degp = _deg_call(dst32)
    dis, xs = _dis_call(degp.T[:NPAD], xp)

    yA = _agg1_call(xs, srcA, dst32)
    HS1 = _l0_call(yA, xs, dis, W[:, 0], b[:, 0][:, None, :])
    y1 = _agg2_call(HS1.reshape(2 * NPAD, D), srcBC, dst16)
    HS2 = _mid_call(y1, HS1, dis, W[:, 1], b[:, 1][:, None, :])
    y2 = _agg2_call(HS2.reshape(2 * NPAD, D), srcBC, dst16)
    ln = _fin_call(y2, HS2, dis, W[:, 2], b[:, 2][:, None, :],
                   gamma[None], beta[None])
    batchs = jnp.ones((_SZ_C, batch.shape[0]), dtype=x.dtype) * batch.astype(x.dtype)
    return (ln, batchs)


# speedup vs baseline: 27.9649x; 1.0033x over previous
import functools

import jax
import jax.numpy as jnp
import numpy as np
from jax import lax
from jax.experimental import pallas as pl
from jax.experimental.pallas import tpu as pltpu
from jax.experimental.pallas import tpu_sc as plsc

N = 10000
NPAD = 10112
D = 128
E = 320000
EB = 64
NBLK1 = 160
NBLK2 = 320
G = 32
EPAD = 32 * NBLK1 * EB
ACCN = 10240
_SZ_C = 2
_SZ_L = 3


def _sc_mesh():
    return plsc.VectorSubcoreMesh(core_axis_name="c", subcore_axis_name="s")


def _make_deg_kernel():
    @functools.partial(
        pl.kernel,
        out_type=jax.ShapeDtypeStruct((2, ACCN), jnp.float32),
        mesh=_sc_mesh(),
        scratch_types=[
            pltpu.VMEM((NBLK1, EB), jnp.int32),
            pltpu.VMEM((EB,), jnp.float32),
            pltpu.VMEM((640,), jnp.float32),
            pltpu.VMEM_SHARED((ACCN,), jnp.float32),
        ],
    )
    def deg_kernel(dst_hbm, degp_hbm, dst_v, ones_v, z_v, acc):
        c = lax.axis_index("c")
        s = lax.axis_index("s")
        w = c * 16 + s
        for j in range(EB // 16):
            ones_v[pl.ds(j * 16, 16)] = jnp.ones((16,), jnp.float32)
        for j in range(640 // 16):
            z_v[pl.ds(j * 16, 16)] = jnp.zeros((16,), jnp.float32)
        pltpu.sync_copy(z_v, acc.at[pl.ds(s * 640, 640)])
        pltpu.sync_copy(dst_hbm.at[w], dst_v)
        plsc.subcore_barrier()

        def body(i, carry):
            pltpu.sync_copy(ones_v, acc.at[dst_v.at[i]], add=True)
            return carry

        lax.fori_loop(0, NBLK1, body, 0)
        plsc.subcore_barrier()
        pltpu.sync_copy(acc.at[pl.ds(s * 640, 640)],
                        degp_hbm.at[c, pl.ds(s * 640, 640)])

    return deg_kernel


def _make_agg_kernel(two_ch):
    nblk = NBLK2 if two_ch else NBLK1
    chunk = nblk * EB

    @functools.partial(
        pl.kernel,
        out_type=jax.ShapeDtypeStruct((2, NPAD, D), jnp.float32),
        mesh=_sc_mesh(),
        scratch_types=[
            pltpu.VMEM((G * EB,), jnp.int32),
            pltpu.VMEM((G * EB,), jnp.int32),
            pltpu.VMEM((G, EB), jnp.int32),
            pltpu.VMEM((G, EB), jnp.int32),
            pltpu.VMEM((4, EB, D), jnp.float32),
            pltpu.VMEM_SHARED((NPAD, D), jnp.float32),
            pltpu.SemaphoreType.DMA,
            pltpu.SemaphoreType.DMA,
            pltpu.SemaphoreType.DMA,
            pltpu.SemaphoreType.DMA,
            pltpu.SemaphoreType.DMA,
            pltpu.SemaphoreType.DMA,
            pltpu.SemaphoreType.DMA,
            pltpu.SemaphoreType.DMA,
            pltpu.SemaphoreType.DMA,
        ],
    )
    def agg_kernel(tbl_hbm, src_hbm, dst_hbm, out_hbm, src_g0, src_g1,
                   dst_g0, dst_g1, rows_v, acc, gsem0, gsem1, gsem2, gsem3,
                   ssem0, ssem1, ssem2, ssem3, isem):
        c = lax.axis_index("c")
        s = lax.axis_index("s")
        w = c * 16 + s
        ngrp = nblk // G
        gsems = (gsem0, gsem1, gsem2, gsem3)
        ssems = (ssem0, ssem1, ssem2, ssem3)
        src_gs = (src_g0, src_g1)
        dst_gs = (dst_g0, dst_g1)

        def zbody(i, carry):
            for j in range(D // 16):
                rows_v[0, i, pl.ds(j * 16, 16)] = jnp.zeros((16,), jnp.float32)
            return carry

        lax.fori_loop(0, EB, zbody, 0)
        base = s * 632
        z0 = rows_v.at[0]
        for k in range(9):
            pltpu.sync_copy(z0, acc.at[pl.ds(base + k * EB, EB)])
        pltpu.sync_copy(z0.at[pl.ds(0, 56)], acc.at[pl.ds(base + 9 * EB, 56)])
        plsc.subcore_barrier()

        def _src_ref(g):
            off = (s if two_ch else w) * chunk
            return src_hbm.at[pl.ds(off + g * (G * EB), G * EB)]

        def _dst_ref(g):
            if two_ch:
                return dst_hbm.at[s, pl.ds(g * G, G)]
            return dst_hbm.at[w, pl.ds(g * G, G)]

        def _idx_start(g, slot):
            pltpu.async_copy(_src_ref(g), src_gs[slot], isem)
            pltpu.async_copy(_dst_ref(g), dst_gs[slot], isem)

        def _idx_wait(g, slot):
            pltpu.make_async_copy(_src_ref(g), src_gs[slot], isem).wait()
            pltpu.make_async_copy(_dst_ref(g), dst_gs[slot], isem).wait()

        def _gref(slot, j, p):
            tbl = tbl_hbm.at[c] if two_ch else tbl_hbm
            return (tbl.at[src_gs[slot].at[pl.ds(j * EB, EB)]],
                    rows_v.at[p], gsems[p])

        def _sref(slot, j, p):
            return (rows_v.at[p], acc.at[dst_gs[slot].at[j]], ssems[p])

        def run_group(slot):
            for k in range(3):
                pltpu.async_copy(*_gref(slot, k, k))
            for j in range(G):
                p = j % 4
                pltpu.make_async_copy(*_gref(slot, j, p)).wait()
                pltpu.async_copy(*_sref(slot, j, p), add=True)
                if j >= 1:
                    pltpu.make_async_copy(*_sref(slot, j - 1, (j - 1) % 4)).wait()
                if j + 3 < G:
                    pltpu.async_copy(*_gref(slot, j + 3, (j + 3) % 4))
            pltpu.make_async_copy(*_sref(slot, G - 1, (G - 1) % 4)).wait()

        npair = ngrp // 2
        _idx_start(0, 0)

        def pair(gi2, carry):
            gA = 2 * gi2
            _idx_wait(gA, 0)
            _idx_start(gA + 1, 1)
            run_group(0)
            _idx_wait(gA + 1, 1)
            _idx_start(jnp.minimum(gA + 2, ngrp - 1), 0)
            run_group(1)
            return carry

        lax.fori_loop(0, npair, pair, 0)
        if ngrp % 2 == 1:
            _idx_wait(ngrp - 1, 0)
            run_group(0)
        else:
            _idx_wait(ngrp - 1, 0)
        plsc.subcore_barrier()
        dbase = s * 632
        pltpu.sync_copy(acc.at[pl.ds(dbase, 632)],
                        out_hbm.at[c, pl.ds(dbase, 632)])

    return agg_kernel


_deg_call = _make_deg_kernel()
_agg1_call = _make_agg_kernel(False)
_agg2_call = _make_agg_kernel(True)


_BRD = 2528
_BRL = 2528
_BRF = 2000


def _dis_body(degp_ref, x_ref, dis_ref, xs_ref):
    deg = degp_ref[:, 0] + degp_ref[:, 1] + 1.0
    dis = lax.rsqrt(deg)[:, None]
    dis_ref[...] = dis
    xs_ref[...] = dis * x_ref[...]


def _dis_call(degp, xp):
    grid = (NPAD // _BRD,)
    return pl.pallas_call(
        _dis_body,
        grid=grid,
        in_specs=[
            pl.BlockSpec((_BRD, 2), lambda i: (i, 0)),
            pl.BlockSpec((_BRD, D), lambda i: (i, 0)),
        ],
        out_specs=[
            pl.BlockSpec((_BRD, 1), lambda i: (i, 0)),
            pl.BlockSpec((_BRD, D), lambda i: (i, 0)),
        ],
        out_shape=[
            jax.ShapeDtypeStruct((NPAD, 1), jnp.float32),
            jax.ShapeDtypeStruct((NPAD, D), jnp.float32),
        ],
    )(degp, xp)


def _l0_body(y_ref, xs_ref, dis_ref, W_ref, b_ref, o_ref):
    dis = dis_ref[...]
    u = dis * (y_ref[0] + y_ref[1] + xs_ref[...])
    for c in range(_SZ_C):
        g = jnp.dot(u, W_ref[c], preferred_element_type=jnp.float32) + b_ref[c]
        h = jnp.maximum(g, 0.0) + g
        o_ref[c] = dis * h


def _l0_call(y, xs, dis, Wl, bl):
    grid = (NPAD // _BRL,)
    return pl.pallas_call(
        _l0_body,
        grid=grid,
        in_specs=[
            pl.BlockSpec((2, _BRL, D), lambda i: (0, i, 0)),
            pl.BlockSpec((_BRL, D), lambda i: (i, 0)),
            pl.BlockSpec((_BRL, 1), lambda i: (i, 0)),
            pl.BlockSpec((2, D, D), lambda i: (0, 0, 0)),
            pl.BlockSpec((2, 1, D), lambda i: (0, 0, 0)),
        ],
        out_specs=pl.BlockSpec((2, _BRL, D), lambda i: (0, i, 0)),
        out_shape=jax.ShapeDtypeStruct((2, NPAD, D), jnp.float32),
    )(y, xs, dis, Wl, bl)


def _mid_body(y_ref, hs_ref, dis_ref, W_ref, b_ref, o_ref):
    dis = dis_ref[...]
    for c in range(_SZ_C):
        v = dis * (y_ref[c] + hs_ref[c])
        g = jnp.dot(v, W_ref[c], preferred_element_type=jnp.float32) + b_ref[c]
        h = jnp.maximum(g, 0.0) + g
        o_ref[c] = dis * h


def _mid_call(y, hs, dis, Wl, bl):
    grid = (NPAD // _BRL,)
    return pl.pallas_call(
        _mid_body,
        grid=grid,
        in_specs=[
            pl.BlockSpec((2, _BRL, D), lambda i: (0, i, 0)),
            pl.BlockSpec((2, _BRL, D), lambda i: (0, i, 0)),
            pl.BlockSpec((_BRL, 1), lambda i: (i, 0)),
            pl.BlockSpec((2, D, D), lambda i: (0, 0, 0)),
            pl.BlockSpec((2, 1, D), lambda i: (0, 0, 0)),
        ],
        out_specs=pl.BlockSpec((2, _BRL, D), lambda i: (0, i, 0)),
        out_shape=jax.ShapeDtypeStruct((2, NPAD, D), jnp.float32),
    )(y, hs, dis, Wl, bl)


def _fin_body(y_ref, hs_ref, dis_ref, W_ref, b_ref, gamma_ref, beta_ref, o_ref):
    dis = dis_ref[...]
    for c in range(_SZ_C):
        v = dis * (y_ref[c] + hs_ref[c])
        g = jnp.dot(v, W_ref[c], preferred_element_type=jnp.float32) + b_ref[c]
        h = jnp.maximum(g, 0.0) + g
        mu = jnp.mean(h, axis=-1, keepdims=True)
        var = jnp.mean((h - mu) ** 2, axis=-1, keepdims=True)
        o_ref[c] = (h - mu) * lax.rsqrt(var + 1e-6) * gamma_ref[...] + beta_ref[...]


def _fin_call(y, hs, dis, Wl, bl, gamma, beta):
    grid = (N // _BRF,)
    return pl.pallas_call(
        _fin_body,
        grid=grid,
        in_specs=[
            pl.BlockSpec((2, _BRF, D), lambda i: (0, i, 0)),
            pl.BlockSpec((2, _BRF, D), lambda i: (0, i, 0)),
            pl.BlockSpec((_BRF, 1), lambda i: (i, 0)),
            pl.BlockSpec((2, D, D), lambda i: (0, 0, 0)),
            pl.BlockSpec((2, 1, D), lambda i: (0, 0, 0)),
            pl.BlockSpec((1, D), lambda i: (0, 0)),
            pl.BlockSpec((1, D), lambda i: (0, 0)),
        ],
        out_specs=pl.BlockSpec((2, _BRF, D), lambda i: (0, i, 0)),
        out_shape=jax.ShapeDtypeStruct((2, N, D), jnp.float32),
    )(y, hs, dis, Wl, bl, gamma, beta)


def kernel(x, edge, batch, W, b, gamma, beta):
    src, dst = edge[0], edge[1]
    npad_e = EPAD - E
    ar = np.arange(npad_e, dtype=np.int32)
    pad_src = jnp.asarray((ar * 997) % N)
    pad_dst = jnp.asarray(N + (ar % (NPAD - N)))
    srcA = jnp.concatenate([src, pad_src])
    dstA = jnp.concatenate([dst, pad_dst])
    dst32 = dstA.reshape(32, NBLK1, EB)
    dst16 = dstA.reshape(16, NBLK2, EB)
    xp = jnp.pad(x, ((0, NPAD - N), (0, 0)))

    degp = _deg_call(dst32)
    dis, xs = _dis_call(degp.T[:NPAD], xp)

    yA = _agg1_call(xs, srcA, dst32)
    HS1 = _l0_call(yA, xs, dis, W[:, 0], b[:, 0][:, None, :])
    y1 = _agg2_call(HS1, srcA, dst16)
    HS2 = _mid_call(y1, HS1, dis, W[:, 1], b[:, 1][:, None, :])
    y2 = _agg2_call(HS2, srcA, dst16)
    ln = _fin_call(y2, HS2, dis, W[:, 2], b[:, 2][:, None, :],
                   gamma[None], beta[None])
    batchs = jnp.ones((_SZ_C, batch.shape[0]), dtype=x.dtype) * batch.astype(x.dtype)
    return (ln, batchs)


# speedup vs baseline: 28.2709x; 1.0109x over previous
import functools

import jax
import jax.numpy as jnp
import numpy as np
from jax import lax
from jax.experimental import pallas as pl
from jax.experimental.pallas import tpu as pltpu
from jax.experimental.pallas import tpu_sc as plsc

N = 10000
NPAD = 10112
D = 128
E = 320000
EB = 64
NBLK1 = 160
NBLK2 = 320
G = 16
EPAD = 32 * NBLK1 * EB
ACCN = 10240
_SZ_C = 2
_SZ_L = 3


def _sc_mesh():
    return plsc.VectorSubcoreMesh(core_axis_name="c", subcore_axis_name="s")


def _make_deg_kernel():
    @functools.partial(
        pl.kernel,
        out_type=jax.ShapeDtypeStruct((2, ACCN), jnp.float32),
        mesh=_sc_mesh(),
        scratch_types=[
            pltpu.VMEM((NBLK1, EB), jnp.int32),
            pltpu.VMEM((EB,), jnp.float32),
            pltpu.VMEM((640,), jnp.float32),
            pltpu.VMEM_SHARED((ACCN,), jnp.float32),
        ],
    )
    def deg_kernel(dst_hbm, degp_hbm, dst_v, ones_v, z_v, acc):
        c = lax.axis_index("c")
        s = lax.axis_index("s")
        w = c * 16 + s
        for j in range(EB // 16):
            ones_v[pl.ds(j * 16, 16)] = jnp.ones((16,), jnp.float32)
        for j in range(640 // 16):
            z_v[pl.ds(j * 16, 16)] = jnp.zeros((16,), jnp.float32)
        pltpu.sync_copy(z_v, acc.at[pl.ds(s * 640, 640)])
        pltpu.sync_copy(dst_hbm.at[w], dst_v)
        plsc.subcore_barrier()

        def body(i, carry):
            pltpu.sync_copy(ones_v, acc.at[dst_v.at[i]], add=True)
            return carry

        lax.fori_loop(0, NBLK1, body, 0)
        plsc.subcore_barrier()
        pltpu.sync_copy(acc.at[pl.ds(s * 640, 640)],
                        degp_hbm.at[c, pl.ds(s * 640, 640)])

    return deg_kernel


def _make_agg_kernel(two_ch):
    nblk = NBLK2 if two_ch else NBLK1
    chunk = nblk * EB

    @functools.partial(
        pl.kernel,
        out_type=jax.ShapeDtypeStruct((2, NPAD, D), jnp.float32),
        mesh=_sc_mesh(),
        scratch_types=[
            pltpu.VMEM((G * EB,), jnp.int32),
            pltpu.VMEM((G * EB,), jnp.int32),
            pltpu.VMEM((G, EB), jnp.int32),
            pltpu.VMEM((G, EB), jnp.int32),
            pltpu.VMEM((4, EB, D), jnp.float32),
            pltpu.VMEM_SHARED((NPAD, D), jnp.float32),
            pltpu.SemaphoreType.DMA,
            pltpu.SemaphoreType.DMA,
            pltpu.SemaphoreType.DMA,
            pltpu.SemaphoreType.DMA,
            pltpu.SemaphoreType.DMA,
            pltpu.SemaphoreType.DMA,
            pltpu.SemaphoreType.DMA,
            pltpu.SemaphoreType.DMA,
            pltpu.SemaphoreType.DMA,
        ],
    )
    def agg_kernel(tbl_hbm, src_hbm, dst_hbm, out_hbm, src_g0, src_g1,
                   dst_g0, dst_g1, rows_v, acc, gsem0, gsem1, gsem2, gsem3,
                   ssem0, ssem1, ssem2, ssem3, isem):
        c = lax.axis_index("c")
        s = lax.axis_index("s")
        w = c * 16 + s
        ngrp = nblk // G
        gsems = (gsem0, gsem1, gsem2, gsem3)
        ssems = (ssem0, ssem1, ssem2, ssem3)
        src_gs = (src_g0, src_g1)
        dst_gs = (dst_g0, dst_g1)

        def zbody(i, carry):
            for j in range(D // 16):
                rows_v[0, i, pl.ds(j * 16, 16)] = jnp.zeros((16,), jnp.float32)
            return carry

        lax.fori_loop(0, EB, zbody, 0)
        base = s * 632
        z0 = rows_v.at[0]
        for k in range(9):
            pltpu.sync_copy(z0, acc.at[pl.ds(base + k * EB, EB)])
        pltpu.sync_copy(z0.at[pl.ds(0, 56)], acc.at[pl.ds(base + 9 * EB, 56)])
        plsc.subcore_barrier()

        def _src_ref(g):
            off = (s if two_ch else w) * chunk
            return src_hbm.at[pl.ds(off + g * (G * EB), G * EB)]

        def _dst_ref(g):
            if two_ch:
                return dst_hbm.at[s, pl.ds(g * G, G)]
            return dst_hbm.at[w, pl.ds(g * G, G)]

        def _idx_start(g, slot):
            pltpu.async_copy(_src_ref(g), src_gs[slot], isem)
            pltpu.async_copy(_dst_ref(g), dst_gs[slot], isem)

        def _idx_wait(g, slot):
            pltpu.make_async_copy(_src_ref(g), src_gs[slot], isem).wait()
            pltpu.make_async_copy(_dst_ref(g), dst_gs[slot], isem).wait()

        def _gref(slot, j, p):
            tbl = tbl_hbm.at[c] if two_ch else tbl_hbm
            return (tbl.at[src_gs[slot].at[pl.ds(j * EB, EB)]],
                    rows_v.at[p], gsems[p])

        def _sref(slot, j, p):
            return (rows_v.at[p], acc.at[dst_gs[slot].at[j]], ssems[p])

        def _blk(jj):
            return jj // G, jj % G, jj % 4

        def run_pair(gA, last):
            for k in range(3):
                slot, j, p = _blk(k)
                pltpu.async_copy(*_gref(slot, j, p))
            for jj in range(2 * G):
                slot, j, p = _blk(jj)
                if jj == G - 3:
                    _idx_wait(gA + 1, 1)
                if jj == G and not last:
                    _idx_start(gA + 2, 0)
                pltpu.make_async_copy(*_gref(slot, j, p)).wait()
                pltpu.async_copy(*_sref(slot, j, p), add=True)
                if jj >= 1:
                    ps, pj, pp = _blk(jj - 1)
                    pltpu.make_async_copy(*_sref(ps, pj, pp)).wait()
                la = jj + 3
                if la < 2 * G:
                    ls, lj, lp = _blk(la)
                    pltpu.async_copy(*_gref(ls, lj, lp))
            ls, lj, lp = _blk(2 * G - 1)
            pltpu.make_async_copy(*_sref(ls, lj, lp)).wait()

        npair = ngrp // 2
        assert ngrp % 2 == 0
        _idx_start(0, 0)

        def pair(gi2, carry):
            gA = 2 * gi2
            _idx_wait(gA, 0)
            _idx_start(gA + 1, 1)
            run_pair(gA, last=False)
            return carry

        lax.fori_loop(0, npair - 1, pair, 0)
        gA = 2 * (npair - 1)
        _idx_wait(gA, 0)
        _idx_start(gA + 1, 1)
        run_pair(gA, last=True)
        plsc.subcore_barrier()
        dbase = s * 632
        pltpu.sync_copy(acc.at[pl.ds(dbase, 632)],
                        out_hbm.at[c, pl.ds(dbase, 632)])

    return agg_kernel


_deg_call = _make_deg_kernel()
_agg1_call = _make_agg_kernel(False)
_agg2_call = _make_agg_kernel(True)


_BRD = 2528
_BRL = 2528
_BRF = 2000


def _dis_body(degp_ref, x_ref, dis_ref, xs_ref):
    deg = degp_ref[:, 0] + degp_ref[:, 1] + 1.0
    dis = lax.rsqrt(deg)[:, None]
    dis_ref[...] = dis
    xs_ref[...] = dis * x_ref[...]


def _dis_call(degp, xp):
    grid = (NPAD // _BRD,)
    return pl.pallas_call(
        _dis_body,
        grid=grid,
        in_specs=[
            pl.BlockSpec((_BRD, 2), lambda i: (i, 0)),
            pl.BlockSpec((_BRD, D), lambda i: (i, 0)),
        ],
        out_specs=[
            pl.BlockSpec((_BRD, 1), lambda i: (i, 0)),
            pl.BlockSpec((_BRD, D), lambda i: (i, 0)),
        ],
        out_shape=[
            jax.ShapeDtypeStruct((NPAD, 1), jnp.float32),
            jax.ShapeDtypeStruct((NPAD, D), jnp.float32),
        ],
    )(degp, xp)


def _l0_body(y_ref, xs_ref, dis_ref, W_ref, b_ref, o_ref):
    dis = dis_ref[...]
    u = dis * (y_ref[0] + y_ref[1] + xs_ref[...])
    for c in range(_SZ_C):
        g = jnp.dot(u, W_ref[c], preferred_element_type=jnp.float32) + b_ref[c]
        h = jnp.maximum(g, 0.0) + g
        o_ref[c] = dis * h


def _l0_call(y, xs, dis, Wl, bl):
    grid = (NPAD // _BRL,)
    return pl.pallas_call(
        _l0_body,
        grid=grid,
        in_specs=[
            pl.BlockSpec((2, _BRL, D), lambda i: (0, i, 0)),
            pl.BlockSpec((_BRL, D), lambda i: (i, 0)),
            pl.BlockSpec((_BRL, 1), lambda i: (i, 0)),
            pl.BlockSpec((2, D, D), lambda i: (0, 0, 0)),
            pl.BlockSpec((2, 1, D), lambda i: (0, 0, 0)),
        ],
        out_specs=pl.BlockSpec((2, _BRL, D), lambda i: (0, i, 0)),
        out_shape=jax.ShapeDtypeStruct((2, NPAD, D), jnp.float32),
    )(y, xs, dis, Wl, bl)


def _mid_body(y_ref, hs_ref, dis_ref, W_ref, b_ref, o_ref):
    dis = dis_ref[...]
    for c in range(_SZ_C):
        v = dis * (y_ref[c] + hs_ref[c])
        g = jnp.dot(v, W_ref[c], preferred_element_type=jnp.float32) + b_ref[c]
        h = jnp.maximum(g, 0.0) + g
        o_ref[c] = dis * h


def _mid_call(y, hs, dis, Wl, bl):
    grid = (NPAD // _BRL,)
    return pl.pallas_call(
        _mid_body,
        grid=grid,
        in_specs=[
            pl.BlockSpec((2, _BRL, D), lambda i: (0, i, 0)),
            pl.BlockSpec((2, _BRL, D), lambda i: (0, i, 0)),
            pl.BlockSpec((_BRL, 1), lambda i: (i, 0)),
            pl.BlockSpec((2, D, D), lambda i: (0, 0, 0)),
            pl.BlockSpec((2, 1, D), lambda i: (0, 0, 0)),
        ],
        out_specs=pl.BlockSpec((2, _BRL, D), lambda i: (0, i, 0)),
        out_shape=jax.ShapeDtypeStruct((2, NPAD, D), jnp.float32),
    )(y, hs, dis, Wl, bl)


def _fin_body(y_ref, hs_ref, dis_ref, W_ref, b_ref, gamma_ref, beta_ref, o_ref):
    dis = dis_ref[...]
    for c in range(_SZ_C):
        v = dis * (y_ref[c] + hs_ref[c])
        g = jnp.dot(v, W_ref[c], preferred_element_type=jnp.float32) + b_ref[c]
        h = jnp.maximum(g, 0.0) + g
        mu = jnp.mean(h, axis=-1, keepdims=True)
        var = jnp.mean((h - mu) ** 2, axis=-1, keepdims=True)
        o_ref[c] = (h - mu) * lax.rsqrt(var + 1e-6) * gamma_ref[...] + beta_ref[...]


def _fin_call(y, hs, dis, Wl, bl, gamma, beta):
    grid = (N // _BRF,)
    return pl.pallas_call(
        _fin_body,
        grid=grid,
        in_specs=[
            pl.BlockSpec((2, _BRF, D), lambda i: (0, i, 0)),
            pl.BlockSpec((2, _BRF, D), lambda i: (0, i, 0)),
            pl.BlockSpec((_BRF, 1), lambda i: (i, 0)),
            pl.BlockSpec((2, D, D), lambda i: (0, 0, 0)),
            pl.BlockSpec((2, 1, D), lambda i: (0, 0, 0)),
            pl.BlockSpec((1, D), lambda i: (0, 0)),
            pl.BlockSpec((1, D), lambda i: (0, 0)),
        ],
        out_specs=pl.BlockSpec((2, _BRF, D), lambda i: (0, i, 0)),
        out_shape=jax.ShapeDtypeStruct((2, N, D), jnp.float32),
    )(y, hs, dis, Wl, bl, gamma, beta)


def kernel(x, edge, batch, W, b, gamma, beta):
    src, dst = edge[0], edge[1]
    npad_e = EPAD - E
    ar = np.arange(npad_e, dtype=np.int32)
    pad_src = jnp.asarray((ar * 997) % N)
    pad_dst = jnp.asarray(N + (ar % (NPAD - N)))
    srcA = jnp.concatenate([src, pad_src])
    dstA = jnp.concatenate([dst, pad_dst])
    dst32 = dstA.reshape(32, NBLK1, EB)
    dst16 = dstA.reshape(16, NBLK2, EB)
    xp = jnp.pad(x, ((0, NPAD - N), (0, 0)))

    degp = _deg_call(dst32)
    dis, xs = _dis_call(degp.T[:NPAD], xp)

    yA = _agg1_call(xs, srcA, dst32)
    HS1 = _l0_call(yA, xs, dis, W[:, 0], b[:, 0][:, None, :])
    y1 = _agg2_call(HS1, srcA, dst16)
    HS2 = _mid_call(y1, HS1, dis, W[:, 1], b[:, 1][:, None, :])
    y2 = _agg2_call(HS2, srcA, dst16)
    ln = _fin_call(y2, HS2, dis, W[:, 2], b[:, 2][:, None, :],
                   gamma[None], beta[None])
    batchs = jnp.ones((_SZ_C, batch.shape[0]), dtype=x.dtype) * batch.astype(x.dtype)
    return (ln, batchs)


# speedup vs baseline: 28.6549x; 1.0136x over previous
import functools

import jax
import jax.numpy as jnp
import numpy as np
from jax import lax
from jax.experimental import pallas as pl
from jax.experimental.pallas import tpu as pltpu
from jax.experimental.pallas import tpu_sc as plsc

N = 10000
NPAD = 10112
D = 128
E = 320000
EB = 64
NBLK1 = 160
NBLK2 = 320
G = 16
EPAD = 32 * NBLK1 * EB
ACCN = 10240
_SZ_C = 2
_SZ_L = 3


def _sc_mesh():
    return plsc.VectorSubcoreMesh(core_axis_name="c", subcore_axis_name="s")


def _make_deg_kernel():
    @functools.partial(
        pl.kernel,
        out_type=jax.ShapeDtypeStruct((2, ACCN), jnp.float32),
        mesh=_sc_mesh(),
        scratch_types=[
            pltpu.VMEM((NBLK1, EB), jnp.int32),
            pltpu.VMEM((EB,), jnp.float32),
            pltpu.VMEM((640,), jnp.float32),
            pltpu.VMEM_SHARED((ACCN,), jnp.float32),
        ],
    )
    def deg_kernel(dst_hbm, degp_hbm, dst_v, ones_v, z_v, acc):
        c = lax.axis_index("c")
        s = lax.axis_index("s")
        w = c * 16 + s
        for j in range(EB // 16):
            ones_v[pl.ds(j * 16, 16)] = jnp.ones((16,), jnp.float32)
        for j in range(640 // 16):
            z_v[pl.ds(j * 16, 16)] = jnp.zeros((16,), jnp.float32)
        pltpu.sync_copy(z_v, acc.at[pl.ds(s * 640, 640)])
        pltpu.sync_copy(dst_hbm.at[w], dst_v)
        plsc.subcore_barrier()

        def body(i, carry):
            pltpu.sync_copy(ones_v, acc.at[dst_v.at[i]], add=True)
            return carry

        lax.fori_loop(0, NBLK1, body, 0)
        plsc.subcore_barrier()
        pltpu.sync_copy(acc.at[pl.ds(s * 640, 640)],
                        degp_hbm.at[c, pl.ds(s * 640, 640)])

    return deg_kernel


def _make_agg_kernel(two_ch):
    nblk = NBLK2 if two_ch else NBLK1
    chunk = nblk * EB
    grp = 2 * G if two_ch else G

    @functools.partial(
        pl.kernel,
        out_type=jax.ShapeDtypeStruct((2, NPAD, D), jnp.float32),
        mesh=_sc_mesh(),
        scratch_types=[
            pltpu.VMEM((grp * EB,), jnp.int32),
            pltpu.VMEM((grp * EB,), jnp.int32),
            pltpu.VMEM((grp, EB), jnp.int32),
            pltpu.VMEM((grp, EB), jnp.int32),
            pltpu.VMEM((4, EB, D), jnp.float32),
            pltpu.VMEM_SHARED((NPAD, D), jnp.float32),
            pltpu.SemaphoreType.DMA,
            pltpu.SemaphoreType.DMA,
            pltpu.SemaphoreType.DMA,
            pltpu.SemaphoreType.DMA,
            pltpu.SemaphoreType.DMA,
            pltpu.SemaphoreType.DMA,
            pltpu.SemaphoreType.DMA,
            pltpu.SemaphoreType.DMA,
            pltpu.SemaphoreType.DMA,
        ],
    )
    def agg_kernel(tbl_hbm, src_hbm, dst_hbm, out_hbm, src_g0, src_g1,
                   dst_g0, dst_g1, rows_v, acc, gsem0, gsem1, gsem2, gsem3,
                   ssem0, ssem1, ssem2, ssem3, isem):
        c = lax.axis_index("c")
        s = lax.axis_index("s")
        w = c * 16 + s
        ngrp = nblk // grp
        gsems = (gsem0, gsem1, gsem2, gsem3)
        ssems = (ssem0, ssem1, ssem2, ssem3)
        src_gs = (src_g0, src_g1)
        dst_gs = (dst_g0, dst_g1)

        def zbody(i, carry):
            for j in range(D // 16):
                rows_v[0, i, pl.ds(j * 16, 16)] = jnp.zeros((16,), jnp.float32)
            return carry

        lax.fori_loop(0, EB, zbody, 0)
        base = s * 632
        z0 = rows_v.at[0]
        for k in range(9):
            pltpu.sync_copy(z0, acc.at[pl.ds(base + k * EB, EB)])
        pltpu.sync_copy(z0.at[pl.ds(0, 56)], acc.at[pl.ds(base + 9 * EB, 56)])
        plsc.subcore_barrier()

        def _src_ref(g):
            off = (s if two_ch else w) * chunk
            return src_hbm.at[pl.ds(off + g * (grp * EB), grp * EB)]

        def _dst_ref(g):
            if two_ch:
                return dst_hbm.at[s, pl.ds(g * grp, grp)]
            return dst_hbm.at[w, pl.ds(g * grp, grp)]

        def _idx_start(g, slot):
            pltpu.async_copy(_src_ref(g), src_gs[slot], isem)
            pltpu.async_copy(_dst_ref(g), dst_gs[slot], isem)

        def _idx_wait(g, slot):
            pltpu.make_async_copy(_src_ref(g), src_gs[slot], isem).wait()
            pltpu.make_async_copy(_dst_ref(g), dst_gs[slot], isem).wait()

        def _gref(slot, j, p):
            tbl = tbl_hbm.at[c] if two_ch else tbl_hbm
            return (tbl.at[src_gs[slot].at[pl.ds(j * EB, EB)]],
                    rows_v.at[p], gsems[p])

        def _sref(slot, j, p):
            return (rows_v.at[p], acc.at[dst_gs[slot].at[j]], ssems[p])

        def _blk(jj):
            return jj // grp, jj % grp, jj % 4

        def run_pair(gA, last):
            for k in range(3):
                slot, j, p = _blk(k)
                pltpu.async_copy(*_gref(slot, j, p))
            for jj in range(2 * grp):
                slot, j, p = _blk(jj)
                if jj == grp - 3:
                    _idx_wait(gA + 1, 1)
                if jj == grp and not last:
                    _idx_start(gA + 2, 0)
                pltpu.make_async_copy(*_gref(slot, j, p)).wait()
                pltpu.async_copy(*_sref(slot, j, p), add=True)
                if jj >= 1:
                    ps, pj, pp = _blk(jj - 1)
                    pltpu.make_async_copy(*_sref(ps, pj, pp)).wait()
                la = jj + 3
                if la < 2 * grp:
                    ls, lj, lp = _blk(la)
                    pltpu.async_copy(*_gref(ls, lj, lp))
            ls, lj, lp = _blk(2 * grp - 1)
            pltpu.make_async_copy(*_sref(ls, lj, lp)).wait()

        npair = ngrp // 2
        assert ngrp % 2 == 0
        _idx_start(0, 0)

        def pair(gi2, carry):
            gA = 2 * gi2
            _idx_wait(gA, 0)
            _idx_start(gA + 1, 1)
            run_pair(gA, last=False)
            return carry

        lax.fori_loop(0, npair - 1, pair, 0)
        gA = 2 * (npair - 1)
        _idx_wait(gA, 0)
        _idx_start(gA + 1, 1)
        run_pair(gA, last=True)
        plsc.subcore_barrier()
        dbase = s * 632
        pltpu.sync_copy(acc.at[pl.ds(dbase, 632)],
                        out_hbm.at[c, pl.ds(dbase, 632)])

    return agg_kernel


_deg_call = _make_deg_kernel()
_agg1_call = _make_agg_kernel(False)
_agg2_call = _make_agg_kernel(True)


_BRD = 2528
_BRL = 2528
_BRF = 2000


def _dis_body(degp_ref, x_ref, dis_ref, xs_ref):
    deg = degp_ref[:, 0] + degp_ref[:, 1] + 1.0
    dis = lax.rsqrt(deg)[:, None]
    dis_ref[...] = dis
    xs_ref[...] = dis * x_ref[...]


def _dis_call(degp, xp):
    grid = (NPAD // _BRD,)
    return pl.pallas_call(
        _dis_body,
        grid=grid,
        in_specs=[
            pl.BlockSpec((_BRD, 2), lambda i: (i, 0)),
            pl.BlockSpec((_BRD, D), lambda i: (i, 0)),
        ],
        out_specs=[
            pl.BlockSpec((_BRD, 1), lambda i: (i, 0)),
            pl.BlockSpec((_BRD, D), lambda i: (i, 0)),
        ],
        out_shape=[
            jax.ShapeDtypeStruct((NPAD, 1), jnp.float32),
            jax.ShapeDtypeStruct((NPAD, D), jnp.float32),
        ],
    )(degp, xp)


def _l0_body(y_ref, xs_ref, dis_ref, W_ref, b_ref, o_ref):
    dis = dis_ref[...]
    u = dis * (y_ref[0] + y_ref[1] + xs_ref[...])
    for c in range(_SZ_C):
        g = jnp.dot(u, W_ref[c], preferred_element_type=jnp.float32) + b_ref[c]
        h = jnp.maximum(g, 0.0) + g
        o_ref[c] = dis * h


def _l0_call(y, xs, dis, Wl, bl):
    grid = (NPAD // _BRL,)
    return pl.pallas_call(
        _l0_body,
        grid=grid,
        in_specs=[
            pl.BlockSpec((2, _BRL, D), lambda i: (0, i, 0)),
            pl.BlockSpec((_BRL, D), lambda i: (i, 0)),
            pl.BlockSpec((_BRL, 1), lambda i: (i, 0)),
            pl.BlockSpec((2, D, D), lambda i: (0, 0, 0)),
            pl.BlockSpec((2, 1, D), lambda i: (0, 0, 0)),
        ],
        out_specs=pl.BlockSpec((2, _BRL, D), lambda i: (0, i, 0)),
        out_shape=jax.ShapeDtypeStruct((2, NPAD, D), jnp.float32),
    )(y, xs, dis, Wl, bl)


def _mid_body(y_ref, hs_ref, dis_ref, W_ref, b_ref, o_ref):
    dis = dis_ref[...]
    for c in range(_SZ_C):
        v = dis * (y_ref[c] + hs_ref[c])
        g = jnp.dot(v, W_ref[c], preferred_element_type=jnp.float32) + b_ref[c]
        h = jnp.maximum(g, 0.0) + g
        o_ref[c] = dis * h


def _mid_call(y, hs, dis, Wl, bl):
    grid = (NPAD // _BRL,)
    return pl.pallas_call(
        _mid_body,
        grid=grid,
        in_specs=[
            pl.BlockSpec((2, _BRL, D), lambda i: (0, i, 0)),
            pl.BlockSpec((2, _BRL, D), lambda i: (0, i, 0)),
            pl.BlockSpec((_BRL, 1), lambda i: (i, 0)),
            pl.BlockSpec((2, D, D), lambda i: (0, 0, 0)),
            pl.BlockSpec((2, 1, D), lambda i: (0, 0, 0)),
        ],
        out_specs=pl.BlockSpec((2, _BRL, D), lambda i: (0, i, 0)),
        out_shape=jax.ShapeDtypeStruct((2, NPAD, D), jnp.float32),
    )(y, hs, dis, Wl, bl)


def _fin_body(y_ref, hs_ref, dis_ref, W_ref, b_ref, gamma_ref, beta_ref, o_ref):
    dis = dis_ref[...]
    for c in range(_SZ_C):
        v = dis * (y_ref[c] + hs_ref[c])
        g = jnp.dot(v, W_ref[c], preferred_element_type=jnp.float32) + b_ref[c]
        h = jnp.maximum(g, 0.0) + g
        mu = jnp.mean(h, axis=-1, keepdims=True)
        var = jnp.mean((h - mu) ** 2, axis=-1, keepdims=True)
        o_ref[c] = (h - mu) * lax.rsqrt(var + 1e-6) * gamma_ref[...] + beta_ref[...]


def _fin_call(y, hs, dis, Wl, bl, gamma, beta):
    grid = (N // _BRF,)
    return pl.pallas_call(
        _fin_body,
        grid=grid,
        in_specs=[
            pl.BlockSpec((2, _BRF, D), lambda i: (0, i, 0)),
            pl.BlockSpec((2, _BRF, D), lambda i: (0, i, 0)),
            pl.BlockSpec((_BRF, 1), lambda i: (i, 0)),
            pl.BlockSpec((2, D, D), lambda i: (0, 0, 0)),
            pl.BlockSpec((2, 1, D), lambda i: (0, 0, 0)),
            pl.BlockSpec((1, D), lambda i: (0, 0)),
            pl.BlockSpec((1, D), lambda i: (0, 0)),
        ],
        out_specs=pl.BlockSpec((2, _BRF, D), lambda i: (0, i, 0)),
        out_shape=jax.ShapeDtypeStruct((2, N, D), jnp.float32),
    )(y, hs, dis, Wl, bl, gamma, beta)


def kernel(x, edge, batch, W, b, gamma, beta):
    src, dst = edge[0], edge[1]
    npad_e = EPAD - E
    ar = np.arange(npad_e, dtype=np.int32)
    pad_src = jnp.asarray((ar * 997) % N)
    pad_dst = jnp.asarray(N + (ar % (NPAD - N)))
    srcA = jnp.concatenate([src, pad_src])
    dstA = jnp.concatenate([dst, pad_dst])
    dst32 = dstA.reshape(32, NBLK1, EB)
    dst16 = dstA.reshape(16, NBLK2, EB)
    xp = jnp.pad(x, ((0, NPAD - N), (0, 0)))

    degp = _deg_call(dst32)
    dis, xs = _dis_call(degp.T[:NPAD], xp)

    yA = _agg1_call(xs, srcA, dst32)
    HS1 = _l0_call(yA, xs, dis, W[:, 0], b[:, 0][:, None, :])
    y1 = _agg2_call(HS1, srcA, dst16)
    HS2 = _mid_call(y1, HS1, dis, W[:, 1], b[:, 1][:, None, :])
    y2 = _agg2_call(HS2, srcA, dst16)
    ln = _fin_call(y2, HS2, dis, W[:, 2], b[:, 2][:, None, :],
                   gamma[None], beta[None])
    batchs = jnp.ones((_SZ_C, batch.shape[0]), dtype=x.dtype) * batch.astype(x.dtype)
    return (ln, batchs)
